# Initial kernel scaffold; baseline (speedup 1.0000x reference)
import jax
import jax.numpy as jnp
from jax.experimental import pallas as pl


def kernel(x, edge_index, edge_values, W, b):
    raise NotImplementedError("write your pallas kernel here")


# speedup vs baseline: 2.5453x; 2.5453x over previous
import functools

import jax
import jax.numpy as jnp
from jax import lax
from jax.experimental import pallas as pl
from jax.experimental.pallas import tpu as pltpu
from jax.experimental.pallas import tpu_sc as plsc

N = 10000
E = 320000
D = 128

NC = 2
NS = 16
CH = 128
CHUNKS = 80
EPT = CH * CHUNKS
E_PAD = EPT * NC * NS
ROWS_PER_TILE = 632
N_PAD = ROWS_PER_TILE * NS
ROW_CHUNKS = (128, 128, 128, 128, 120)


def _mm_body(x_ref, w_ref, o_ref):
    o_ref[...] = jnp.dot(x_ref[...], w_ref[...],
                         preferred_element_type=jnp.float32)


def _matmul(x, W):
    return pl.pallas_call(
        _mm_body,
        grid=(10,),
        in_specs=[
            pl.BlockSpec((N // 10, D), lambda i: (i, 0)),
            pl.BlockSpec((D, D), lambda i: (0, 0)),
        ],
        out_specs=pl.BlockSpec((N // 10, D), lambda i: (i, 0)),
        out_shape=jax.ShapeDtypeStruct((N, D), jnp.float32),
    )(x, W)


def _sc_body(support_hbm, row_hbm, col_hbm, ev_hbm, out_hbm,
             colv, rowv, evv, rows_v, acc, sem):
    c = lax.axis_index("c")
    s = lax.axis_index("s")

    zero = jnp.zeros((16,), jnp.float32)

    def zbody(i, carry):
        for h in range(D // 16):
            rows_v[i, pl.ds(h * 16, 16)] = zero
        return carry

    lax.fori_loop(0, CH, zbody, 0)
    off = 0
    for nrows in ROW_CHUNKS:
        pltpu.sync_copy(rows_v.at[pl.ds(0, nrows)],
                        acc.at[pl.ds(s * ROWS_PER_TILE + off, nrows)])
        off += nrows
    plsc.subcore_barrier()

    base0 = (c * NS + s) * EPT

    def body(k, carry):
        base = base0 + k * CH
        pltpu.sync_copy(col_hbm.at[pl.ds(base, CH)], colv)
        pltpu.sync_copy(row_hbm.at[pl.ds(base, CH)], rowv)
        pltpu.sync_copy(ev_hbm.at[pl.ds(base, CH)], evv)
        pltpu.async_copy(support_hbm.at[colv], rows_v, sem).wait()
        for g in range(CH // 16):
            evg = evv[pl.ds(g * 16, 16)]
            for j in range(16):
                e = g * 16 + j
                sc = lax.gather(
                    evg, jnp.full((16, 1), j, jnp.int32),
                    lax.GatherDimensionNumbers(
                        offset_dims=(), collapsed_slice_dims=(0,),
                        start_index_map=(0,)),
                    slice_sizes=(1,),
                    mode=lax.GatherScatterMode.PROMISE_IN_BOUNDS)
                for h in range(D // 16):
                    rows_v[e, pl.ds(h * 16, 16)] = (
                        rows_v[e, pl.ds(h * 16, 16)] * sc)
        pltpu.sync_copy(rows_v, acc.at[rowv], add=True)
        return carry

    lax.fori_loop(0, CHUNKS, body, 0)
    plsc.subcore_barrier()

    off = 0
    for nrows in ROW_CHUNKS:
        r0 = s * ROWS_PER_TILE + off
        pltpu.sync_copy(acc.at[pl.ds(r0, nrows)],
                        out_hbm.at[c, pl.ds(r0, nrows)])
        off += nrows


_sc_scatter = functools.partial(
    pl.kernel,
    out_type=jax.ShapeDtypeStruct((NC, N_PAD, D), jnp.float32),
    mesh=plsc.VectorSubcoreMesh(core_axis_name="c", subcore_axis_name="s"),
    scratch_types=[
        pltpu.VMEM((CH,), jnp.int32),
        pltpu.VMEM((CH,), jnp.int32),
        pltpu.VMEM((CH,), jnp.float32),
        pltpu.VMEM((CH, D), jnp.float32),
        pltpu.VMEM_SHARED((N_PAD, D), jnp.float32),
        pltpu.SemaphoreType.DMA,
    ],
)(_sc_body)


def _comb_body(p_ref, b_ref, o_ref):
    o_ref[...] = p_ref[0] + p_ref[1] + b_ref[...]


def _combine(parts, b):
    return pl.pallas_call(
        _comb_body,
        grid=(10,),
        in_specs=[
            pl.BlockSpec((NC, N // 10, D), lambda i: (0, i, 0)),
            pl.BlockSpec((1, D), lambda i: (0, 0)),
        ],
        out_specs=pl.BlockSpec((N // 10, D), lambda i: (i, 0)),
        out_shape=jax.ShapeDtypeStruct((N, D), jnp.float32),
    )(parts, b.reshape(1, D))


def kernel(x, edge_index, edge_values, W, b):
    support = _matmul(x, W)
    pad = E_PAD - E
    row = jnp.pad(edge_index[0], (0, pad))
    col = jnp.pad(edge_index[1], (0, pad))
    ev = jnp.pad(edge_values, (0, pad))
    parts = _sc_scatter(support, row, col, ev)
    return _combine(parts[:, :N], b)


# speedup vs baseline: 3.6022x; 1.4152x over previous
import functools

import jax
import jax.numpy as jnp
from jax import lax
from jax.experimental import pallas as pl
from jax.experimental.pallas import tpu as pltpu
from jax.experimental.pallas import tpu_sc as plsc

N = 10000
E = 320000
D = 128

NC = 2
NS = 16
CH = 128
CHUNKS = 80
EPT = CH * CHUNKS
E_PAD = EPT * NC * NS
ROWS_PER_TILE = 632
N_PAD = ROWS_PER_TILE * NS
ROW_CHUNKS = (128, 128, 128, 128, 120)


def _mm_body(x_ref, w_ref, o_ref):
    o_ref[...] = jnp.dot(x_ref[...], w_ref[...],
                         preferred_element_type=jnp.float32)


def _matmul(x, W):
    return pl.pallas_call(
        _mm_body,
        grid=(10,),
        in_specs=[
            pl.BlockSpec((N // 10, D), lambda i: (i, 0)),
            pl.BlockSpec((D, D), lambda i: (0, 0)),
        ],
        out_specs=pl.BlockSpec((N // 10, D), lambda i: (i, 0)),
        out_shape=jax.ShapeDtypeStruct((N, D), jnp.float32),
    )(x, W)


def _bcast16(vec, j):
    return lax.gather(
        vec, jnp.full((16, 1), j, jnp.int32),
        lax.GatherDimensionNumbers(
            offset_dims=(), collapsed_slice_dims=(0,),
            start_index_map=(0,)),
        slice_sizes=(1,),
        mode=lax.GatherScatterMode.PROMISE_IN_BOUNDS)


def _scale_rows(buf, ev1, c):
    def gbody(g, carry):
        evg = ev1[pl.ds(c * CH + g * 16, 16)]
        for j in range(16):
            sc = _bcast16(evg, j)
            e = g * 16 + j
            for h in range(D // 16):
                buf[e, pl.ds(h * 16, 16)] = buf[e, pl.ds(h * 16, 16)] * sc
        return carry

    lax.fori_loop(0, CH // 16, gbody, 0)


def _sc_body(support_hbm, row_hbm, col_hbm, ev_hbm, out_hbm,
             colb0, colb1, colb2, colb3, rowb0, rowb1, rowb2, rowb3,
             ev1, bufA, bufB, acc,
             semI0, semI1, semI2, semI3, semG0, semG1, semS0, semS1):
    c = lax.axis_index("c")
    s = lax.axis_index("s")
    wid = c * NS + s
    cols = (colb0, colb1, colb2, colb3)
    rows_ = (rowb0, rowb1, rowb2, rowb3)
    bufs = (bufA, bufB)
    semI = (semI0, semI1, semI2, semI3)
    semG = (semG0, semG1)
    semS = (semS0, semS1)
    base0 = wid * EPT

    def idx_issue(j, jm4):
        pltpu.async_copy(col_hbm.at[pl.ds(base0 + j * CH, CH)],
                         cols[jm4], semI[jm4])
        pltpu.async_copy(row_hbm.at[pl.ds(base0 + j * CH, CH)],
                         rows_[jm4], semI[jm4])

    def idx_wait(jm4):
        pltpu.make_async_copy(col_hbm.at[pl.ds(base0, CH)],
                              cols[jm4], semI[jm4]).wait()
        pltpu.make_async_copy(row_hbm.at[pl.ds(base0, CH)],
                              rows_[jm4], semI[jm4]).wait()

    def scat_wait(p2, r4):
        pltpu.make_async_copy(bufs[p2], acc.at[rows_[r4]], semS[p2]).wait()

    zero = jnp.zeros((16,), jnp.float32)

    def zbody(i, carry):
        for h in range(D // 16):
            bufA[i, pl.ds(h * 16, 16)] = zero
        return carry

    lax.fori_loop(0, CH, zbody, 0)
    off = 0
    for nrows in ROW_CHUNKS:
        pltpu.sync_copy(bufA.at[pl.ds(0, nrows)],
                        acc.at[pl.ds(s * ROWS_PER_TILE + off, nrows)])
        off += nrows

    pltpu.sync_copy(ev_hbm.at[pl.ds(base0, EPT)], ev1)
    plsc.subcore_barrier()
    idx_issue(0, 0)
    idx_issue(1, 1)
    idx_issue(2, 2)
    idx_wait(0)
    pltpu.async_copy(support_hbm.at[cols[0]], bufs[0], semG[0])

    def body(k2, carry):
        for cc in range(4):
            p4, p2 = (cc + 1) % 4, (cc + 1) % 2
            q4, q2 = cc, cc % 2
            r4 = (cc + 3) % 4
            cdyn = 4 * k2 + cc + 1

            if cc == 0:
                @pl.when(k2 >= 1)
                def _():
                    scat_wait(p2, r4)
            else:
                scat_wait(p2, r4)

            if cc == 0:
                idx_issue(cdyn + 2, r4)
            else:
                @pl.when(k2 < CHUNKS // 4 - 1)
                def _():
                    idx_issue(cdyn + 2, r4)

            def _gather():
                idx_wait(p4)
                pltpu.async_copy(support_hbm.at[cols[p4]], bufs[p2],
                                 semG[p2])

            if cc == 3:
                @pl.when(k2 < CHUNKS // 4 - 1)
                def _():
                    _gather()
            else:
                _gather()

            pltpu.make_async_copy(support_hbm.at[cols[q4]], bufs[q2],
                                  semG[q2]).wait()
            _scale_rows(bufs[q2], ev1, 4 * k2 + cc)
            pltpu.async_copy(bufs[q2], acc.at[rows_[q4]], semS[q2],
                             add=True)
        return carry

    lax.fori_loop(0, CHUNKS // 4, body, 0)
    scat_wait(1, 3)
    plsc.subcore_barrier()

    off = 0
    for nrows in ROW_CHUNKS:
        r0 = s * ROWS_PER_TILE + off
        pltpu.sync_copy(acc.at[pl.ds(r0, nrows)],
                        out_hbm.at[c, pl.ds(r0, nrows)])
        off += nrows


_sc_scatter = functools.partial(
    pl.kernel,
    out_type=jax.ShapeDtypeStruct((NC, N_PAD, D), jnp.float32),
    mesh=plsc.VectorSubcoreMesh(core_axis_name="c", subcore_axis_name="s"),
    scratch_types=[
        pltpu.VMEM((CH,), jnp.int32),
        pltpu.VMEM((CH,), jnp.int32),
        pltpu.VMEM((CH,), jnp.int32),
        pltpu.VMEM((CH,), jnp.int32),
        pltpu.VMEM((CH,), jnp.int32),
        pltpu.VMEM((CH,), jnp.int32),
        pltpu.VMEM((CH,), jnp.int32),
        pltpu.VMEM((CH,), jnp.int32),
        pltpu.VMEM((EPT,), jnp.float32),
        pltpu.VMEM((CH, D), jnp.float32),
        pltpu.VMEM((CH, D), jnp.float32),
        pltpu.VMEM_SHARED((N_PAD, D), jnp.float32),
        pltpu.SemaphoreType.DMA,
        pltpu.SemaphoreType.DMA,
        pltpu.SemaphoreType.DMA,
        pltpu.SemaphoreType.DMA,
        pltpu.SemaphoreType.DMA,
        pltpu.SemaphoreType.DMA,
        pltpu.SemaphoreType.DMA,
        pltpu.SemaphoreType.DMA,
    ],
)(_sc_body)


def _comb_body(p_ref, b_ref, o_ref):
    o_ref[...] = p_ref[0] + p_ref[1] + b_ref[...]


def _combine(parts, b):
    return pl.pallas_call(
        _comb_body,
        grid=(10,),
        in_specs=[
            pl.BlockSpec((NC, N // 10, D), lambda i: (0, i, 0)),
            pl.BlockSpec((1, D), lambda i: (0, 0)),
        ],
        out_specs=pl.BlockSpec((N // 10, D), lambda i: (i, 0)),
        out_shape=jax.ShapeDtypeStruct((N, D), jnp.float32),
    )(parts, b.reshape(1, D))


def kernel(x, edge_index, edge_values, W, b):
    support = _matmul(x, W)
    pad = E_PAD - E
    row = jnp.pad(edge_index[0], (0, pad))
    col = jnp.pad(edge_index[1], (0, pad))
    ev = jnp.pad(edge_values, (0, pad))
    parts = _sc_scatter(support, row, col, ev)
    return _combine(parts[:, :N], b)


# speedup vs baseline: 10.0334x; 2.7854x over previous
import functools

import jax
import jax.numpy as jnp
from jax import lax
from jax.experimental import pallas as pl
from jax.experimental.pallas import tpu as pltpu
from jax.experimental.pallas import tpu_sc as plsc

N = 10000
E = 320000
D = 128

NC = 2
NS = 16
CH = 128
CHUNKS = 80
EPT = CH * CHUNKS
E_PAD = EPT * NC * NS
ROWS_PER_TILE = 632
N_PAD = ROWS_PER_TILE * NS
ROW_CHUNKS = (128, 128, 128, 128, 120)


def _mm_body(x_ref, w_ref, o_ref):
    o_ref[...] = jnp.dot(x_ref[...], w_ref[...],
                         preferred_element_type=jnp.float32)


def _matmul(x, W):
    return pl.pallas_call(
        _mm_body,
        grid=(10,),
        in_specs=[
            pl.BlockSpec((N // 10, D), lambda i: (i, 0)),
            pl.BlockSpec((D, D), lambda i: (0, 0)),
        ],
        out_specs=pl.BlockSpec((N // 10, D), lambda i: (i, 0)),
        out_shape=jax.ShapeDtypeStruct((N, D), jnp.float32),
    )(x, W)


def _bcast16(vec, j):
    return lax.gather(
        vec, jnp.full((16, 1), j, jnp.int32),
        lax.GatherDimensionNumbers(
            offset_dims=(), collapsed_slice_dims=(0,),
            start_index_map=(0,)),
        slice_sizes=(1,),
        mode=lax.GatherScatterMode.PROMISE_IN_BOUNDS)


def _scale_rows(buf, ev1, c):
    def gbody(g, carry):
        evg = ev1[pl.ds(c * CH + g * 16, 16)]
        for j in range(16):
            sc = _bcast16(evg, j)
            e = g * 16 + j
            for h in range(D // 16):
                buf[e, pl.ds(h * 16, 16)] = buf[e, pl.ds(h * 16, 16)] * sc
        return carry

    lax.fori_loop(0, CH // 16, gbody, 0)


def _sc_body(support_hbm, row_hbm, col_hbm, ev_hbm, out_hbm,
             colb0, colb1, colb2, colb3, rowb0, rowb1, rowb2, rowb3,
             ev1, bufA, bufB, acc,
             semI0, semI1, semI2, semI3, semG0, semG1, semS0, semS1):
    c = lax.axis_index("c")
    s = lax.axis_index("s")
    wid = c * NS + s
    cols = (colb0, colb1, colb2, colb3)
    rows_ = (rowb0, rowb1, rowb2, rowb3)
    bufs = (bufA, bufB)
    semI = (semI0, semI1, semI2, semI3)
    semG = (semG0, semG1)
    semS = (semS0, semS1)
    base0 = wid * EPT

    def idx_issue(j, jm4):
        pltpu.async_copy(col_hbm.at[pl.ds(base0 + j * CH, CH)],
                         cols[jm4], semI[jm4])
        pltpu.async_copy(row_hbm.at[pl.ds(base0 + j * CH, CH)],
                         rows_[jm4], semI[jm4])

    def idx_wait(jm4):
        pltpu.make_async_copy(col_hbm.at[pl.ds(base0, CH)],
                              cols[jm4], semI[jm4]).wait()
        pltpu.make_async_copy(row_hbm.at[pl.ds(base0, CH)],
                              rows_[jm4], semI[jm4]).wait()

    def scat_wait(p2, r4):
        pltpu.make_async_copy(bufs[p2], acc.at[rows_[r4]], semS[p2]).wait()

    zero = jnp.zeros((16,), jnp.float32)

    def zbody(i, carry):
        for h in range(D // 16):
            bufA[i, pl.ds(h * 16, 16)] = zero
        return carry

    lax.fori_loop(0, CH, zbody, 0)
    off = 0
    for nrows in ROW_CHUNKS:
        pltpu.sync_copy(bufA.at[pl.ds(0, nrows)],
                        acc.at[pl.ds(s * ROWS_PER_TILE + off, nrows)])
        off += nrows

    pltpu.sync_copy(ev_hbm.at[pl.ds(base0, EPT)], ev1)
    plsc.subcore_barrier()
    idx_issue(0, 0)
    idx_issue(1, 1)
    idx_issue(2, 2)
    idx_wait(0)
    pltpu.async_copy(support_hbm.at[cols[0]], bufs[0], semG[0])

    def body(k2, carry):
        for cc in range(4):
            p4, p2 = (cc + 1) % 4, (cc + 1) % 2
            q4, q2 = cc, cc % 2
            r4 = (cc + 3) % 4
            cdyn = 4 * k2 + cc + 1

            if cc == 0:
                @pl.when(k2 >= 1)
                def _():
                    scat_wait(p2, r4)
            else:
                scat_wait(p2, r4)

            if cc == 0:
                idx_issue(cdyn + 2, r4)
            else:
                @pl.when(k2 < CHUNKS // 4 - 1)
                def _():
                    idx_issue(cdyn + 2, r4)

            def _gather():
                idx_wait(p4)
                pltpu.async_copy(support_hbm.at[cols[p4]], bufs[p2],
                                 semG[p2])

            if cc == 3:
                @pl.when(k2 < CHUNKS // 4 - 1)
                def _():
                    _gather()
            else:
                _gather()

            pltpu.make_async_copy(support_hbm.at[cols[q4]], bufs[q2],
                                  semG[q2]).wait()
            _scale_rows(bufs[q2], ev1, 4 * k2 + cc)
            pltpu.async_copy(bufs[q2], acc.at[rows_[q4]], semS[q2],
                             add=True)
        return carry

    lax.fori_loop(0, CHUNKS // 4, body, 0)
    scat_wait(1, 3)
    plsc.subcore_barrier()

    off = 0
    for nrows in ROW_CHUNKS:
        r0 = s * ROWS_PER_TILE + off
        pltpu.sync_copy(acc.at[pl.ds(r0, nrows)],
                        out_hbm.at[c, pl.ds(r0, nrows)])
        off += nrows


_sc_scatter = functools.partial(
    pl.kernel,
    out_type=jax.ShapeDtypeStruct((NC, N_PAD, D), jnp.float32),
    mesh=plsc.VectorSubcoreMesh(core_axis_name="c", subcore_axis_name="s"),
    scratch_types=[
        pltpu.VMEM((CH,), jnp.int32),
        pltpu.VMEM((CH,), jnp.int32),
        pltpu.VMEM((CH,), jnp.int32),
        pltpu.VMEM((CH,), jnp.int32),
        pltpu.VMEM((CH,), jnp.int32),
        pltpu.VMEM((CH,), jnp.int32),
        pltpu.VMEM((CH,), jnp.int32),
        pltpu.VMEM((CH,), jnp.int32),
        pltpu.VMEM((EPT,), jnp.float32),
        pltpu.VMEM((CH, D), jnp.float32),
        pltpu.VMEM((CH, D), jnp.float32),
        pltpu.VMEM_SHARED((N_PAD, D), jnp.float32),
        pltpu.SemaphoreType.DMA,
        pltpu.SemaphoreType.DMA,
        pltpu.SemaphoreType.DMA,
        pltpu.SemaphoreType.DMA,
        pltpu.SemaphoreType.DMA,
        pltpu.SemaphoreType.DMA,
        pltpu.SemaphoreType.DMA,
        pltpu.SemaphoreType.DMA,
    ],
)(_sc_body)


def _comb_body(p_ref, b_ref, o_ref):
    o_ref[...] = p_ref[0] + p_ref[1] + b_ref[...]


def _combine(parts, b):
    return pl.pallas_call(
        _comb_body,
        grid=(10,),
        in_specs=[
            pl.BlockSpec((NC, N // 10, D), lambda i: (0, i, 0)),
            pl.BlockSpec((1, D), lambda i: (0, 0)),
        ],
        out_specs=pl.BlockSpec((N // 10, D), lambda i: (i, 0)),
        out_shape=jax.ShapeDtypeStruct((N, D), jnp.float32),
    )(parts, b.reshape(1, D))


def kernel(x, edge_index, edge_values, W, b):
    support = _matmul(x, W)
    pad = E_PAD - E
    ar = jnp.arange(pad, dtype=jnp.int32)
    row = jnp.concatenate([edge_index[0], N + ar % (N_PAD - N)])
    col = jnp.concatenate([edge_index[1], ar % N])
    ev = jnp.pad(edge_values, (0, pad))
    parts = _sc_scatter(support, row, col, ev)
    return _combine(parts[:, :N], b)


# speedup vs baseline: 11.0098x; 1.0973x over previous
import functools

import jax
import jax.numpy as jnp
import numpy as np
from jax import lax
from jax.experimental import pallas as pl
from jax.experimental.pallas import tpu as pltpu
from jax.experimental.pallas import tpu_sc as plsc

N = 10000
E = 320000
D = 128

NC = 2
NS = 16
CH = 128
CHUNKS = 80
EPT = CH * CHUNKS
E_PAD = EPT * NC * NS
ROWS_PER_TILE = 632
N_PAD = ROWS_PER_TILE * NS
ROW_CHUNKS = (128, 128, 128, 128, 120)


def _mm_body(x_ref, w_ref, o_ref):
    o_ref[...] = jnp.dot(x_ref[...], w_ref[...],
                         preferred_element_type=jnp.float32)


def _matmul(x, W):
    return pl.pallas_call(
        _mm_body,
        grid=(5,),
        in_specs=[
            pl.BlockSpec((N // 5, D), lambda i: (i, 0)),
            pl.BlockSpec((D, D), lambda i: (0, 0)),
        ],
        out_specs=pl.BlockSpec((N // 5, D), lambda i: (i, 0)),
        out_shape=jax.ShapeDtypeStruct((N, D), jnp.float32),
    )(x, W)


def _bcast16(vec, j):
    return lax.gather(
        vec, jnp.full((16, 1), j, jnp.int32),
        lax.GatherDimensionNumbers(
            offset_dims=(), collapsed_slice_dims=(0,),
            start_index_map=(0,)),
        slice_sizes=(1,),
        mode=lax.GatherScatterMode.PROMISE_IN_BOUNDS)


def _scale_rows(buf, ev1, c):
    def gbody(g, carry):
        evg = ev1[pl.ds(c * CH + g * 16, 16)]
        for j in range(16):
            sc = _bcast16(evg, j)
            e = g * 16 + j
            for h in range(D // 16):
                buf[e, pl.ds(h * 16, 16)] = buf[e, pl.ds(h * 16, 16)] * sc
        return carry

    lax.fori_loop(0, CH // 16, gbody, 0)


def _sc_body(support_hbm, row_hbm, col_hbm, ev_hbm, out_hbm,
             colb0, colb1, colb2, colb3, rowb0, rowb1, rowb2, rowb3,
             ev1, bufA, bufB, acc,
             semI0, semI1, semI2, semI3, semG0, semG1, semS0, semS1):
    c = lax.axis_index("c")
    s = lax.axis_index("s")
    wid = c * NS + s
    cols = (colb0, colb1, colb2, colb3)
    rows_ = (rowb0, rowb1, rowb2, rowb3)
    bufs = (bufA, bufB)
    semI = (semI0, semI1, semI2, semI3)
    semG = (semG0, semG1)
    semS = (semS0, semS1)
    base0 = wid * EPT

    def idx_issue(j, jm4):
        pltpu.async_copy(col_hbm.at[pl.ds(base0 + j * CH, CH)],
                         cols[jm4], semI[jm4])
        pltpu.async_copy(row_hbm.at[pl.ds(base0 + j * CH, CH)],
                         rows_[jm4], semI[jm4])

    def idx_wait(jm4):
        pltpu.make_async_copy(col_hbm.at[pl.ds(base0, CH)],
                              cols[jm4], semI[jm4]).wait()
        pltpu.make_async_copy(row_hbm.at[pl.ds(base0, CH)],
                              rows_[jm4], semI[jm4]).wait()

    def scat_wait(p2, r4):
        pltpu.make_async_copy(bufs[p2], acc.at[rows_[r4]], semS[p2]).wait()

    zero = jnp.zeros((16,), jnp.float32)

    def zbody(i, carry):
        for h in range(D // 16):
            bufA[i, pl.ds(h * 16, 16)] = zero
        return carry

    lax.fori_loop(0, CH, zbody, 0)
    off = 0
    for nrows in ROW_CHUNKS:
        pltpu.sync_copy(bufA.at[pl.ds(0, nrows)],
                        acc.at[pl.ds(s * ROWS_PER_TILE + off, nrows)])
        off += nrows

    pltpu.sync_copy(ev_hbm.at[pl.ds(base0, EPT)], ev1)
    plsc.subcore_barrier()
    idx_issue(0, 0)
    idx_issue(1, 1)
    idx_issue(2, 2)
    idx_wait(0)
    pltpu.async_copy(support_hbm.at[cols[0]], bufs[0], semG[0])

    def body(k2, carry):
        for cc in range(4):
            p4, p2 = (cc + 1) % 4, (cc + 1) % 2
            q4, q2 = cc, cc % 2
            r4 = (cc + 3) % 4
            cdyn = 4 * k2 + cc + 1

            if cc == 0:
                @pl.when(k2 >= 1)
                def _():
                    scat_wait(p2, r4)
            else:
                scat_wait(p2, r4)

            if cc == 0:
                idx_issue(cdyn + 2, r4)
            else:
                @pl.when(k2 < CHUNKS // 4 - 1)
                def _():
                    idx_issue(cdyn + 2, r4)

            def _gather():
                idx_wait(p4)
                pltpu.async_copy(support_hbm.at[cols[p4]], bufs[p2],
                                 semG[p2])

            if cc == 3:
                @pl.when(k2 < CHUNKS // 4 - 1)
                def _():
                    _gather()
            else:
                _gather()

            pltpu.make_async_copy(support_hbm.at[cols[q4]], bufs[q2],
                                  semG[q2]).wait()
            _scale_rows(bufs[q2], ev1, 4 * k2 + cc)
            pltpu.async_copy(bufs[q2], acc.at[rows_[q4]], semS[q2],
                             add=True)
        return carry

    lax.fori_loop(0, CHUNKS // 4, body, 0)
    scat_wait(1, 3)
    plsc.subcore_barrier()

    off = 0
    for nrows in ROW_CHUNKS:
        r0 = s * ROWS_PER_TILE + off
        pltpu.sync_copy(acc.at[pl.ds(r0, nrows)],
                        out_hbm.at[c, pl.ds(r0, nrows)])
        off += nrows


_sc_scatter = functools.partial(
    pl.kernel,
    out_type=jax.ShapeDtypeStruct((NC, N_PAD, D), jnp.float32),
    mesh=plsc.VectorSubcoreMesh(core_axis_name="c", subcore_axis_name="s"),
    scratch_types=[
        pltpu.VMEM((CH,), jnp.int32),
        pltpu.VMEM((CH,), jnp.int32),
        pltpu.VMEM((CH,), jnp.int32),
        pltpu.VMEM((CH,), jnp.int32),
        pltpu.VMEM((CH,), jnp.int32),
        pltpu.VMEM((CH,), jnp.int32),
        pltpu.VMEM((CH,), jnp.int32),
        pltpu.VMEM((CH,), jnp.int32),
        pltpu.VMEM((EPT,), jnp.float32),
        pltpu.VMEM((CH, D), jnp.float32),
        pltpu.VMEM((CH, D), jnp.float32),
        pltpu.VMEM_SHARED((N_PAD, D), jnp.float32),
        pltpu.SemaphoreType.DMA,
        pltpu.SemaphoreType.DMA,
        pltpu.SemaphoreType.DMA,
        pltpu.SemaphoreType.DMA,
        pltpu.SemaphoreType.DMA,
        pltpu.SemaphoreType.DMA,
        pltpu.SemaphoreType.DMA,
        pltpu.SemaphoreType.DMA,
    ],
)(_sc_body)


def _comb_body(p_ref, b_ref, o_ref):
    o_ref[...] = p_ref[0] + p_ref[1] + b_ref[...]


def _combine(parts, b):
    return pl.pallas_call(
        _comb_body,
        grid=(10,),
        in_specs=[
            pl.BlockSpec((NC, N // 10, D), lambda i: (0, i, 0)),
            pl.BlockSpec((1, D), lambda i: (0, 0)),
        ],
        out_specs=pl.BlockSpec((N // 10, D), lambda i: (i, 0)),
        out_shape=jax.ShapeDtypeStruct((N, D), jnp.float32),
    )(parts, b.reshape(1, D))


_PAD_AR = np.arange(E_PAD - E, dtype=np.int32)
_PAD_RC = jnp.asarray(np.stack([N + _PAD_AR % (N_PAD - N), _PAD_AR % N]))


def kernel(x, edge_index, edge_values, W, b):
    support = _matmul(x, W)
    rc = jnp.concatenate([edge_index, _PAD_RC], axis=1)
    ev = jnp.pad(edge_values, (0, E_PAD - E))
    parts = _sc_scatter(support, rc[0], rc[1], ev)
    return _combine(parts, b)


# speedup vs baseline: 11.6034x; 1.0539x over previous
import functools

import jax
import jax.numpy as jnp
from jax import lax
from jax.experimental import pallas as pl
from jax.experimental.pallas import tpu as pltpu
from jax.experimental.pallas import tpu_sc as plsc

N = 10000
E = 320000
D = 128

NC = 2
NS = 16
CH = 128
CHUNKS = 80
EPT = CH * CHUNKS
FULL_TILES = E // EPT
TAIL_E = E - FULL_TILES * EPT
TAIL_CHUNKS = TAIL_E // CH
ROWS_PER_TILE = 632
N_PAD = ROWS_PER_TILE * NS
ROW_CHUNKS = (128, 128, 128, 128, 120)


def _mm_body(x_ref, w_ref, o_ref):
    o_ref[...] = jnp.dot(x_ref[...], w_ref[...],
                         preferred_element_type=jnp.float32)


def _matmul(x, W):
    return pl.pallas_call(
        _mm_body,
        grid=(5,),
        in_specs=[
            pl.BlockSpec((N // 5, D), lambda i: (i, 0)),
            pl.BlockSpec((D, D), lambda i: (0, 0)),
        ],
        out_specs=pl.BlockSpec((N // 5, D), lambda i: (i, 0)),
        out_shape=jax.ShapeDtypeStruct((N, D), jnp.float32),
    )(x, W)


def _bcast16(vec, j):
    return lax.gather(
        vec, jnp.full((16, 1), j, jnp.int32),
        lax.GatherDimensionNumbers(
            offset_dims=(), collapsed_slice_dims=(0,),
            start_index_map=(0,)),
        slice_sizes=(1,),
        mode=lax.GatherScatterMode.PROMISE_IN_BOUNDS)


def _scale_rows(buf, ev1, c):
    def gbody(g, carry):
        evg = ev1[pl.ds(c * CH + g * 16, 16)]
        for j in range(16):
            sc = _bcast16(evg, j)
            e = g * 16 + j
            for h in range(D // 16):
                buf[e, pl.ds(h * 16, 16)] = buf[e, pl.ds(h * 16, 16)] * sc
        return carry

    lax.fori_loop(0, CH // 16, gbody, 0)


def _sc_body(support_hbm, eidx_hbm, ev_hbm, out_hbm,
             colb0, colb1, colb2, colb3, rowb0, rowb1, rowb2, rowb3,
             ev1, bufA, bufB, acc,
             semI0, semI1, semI2, semI3, semG0, semG1, semS0, semS1):
    c = lax.axis_index("c")
    s = lax.axis_index("s")
    wid = c * NS + s
    cols = (colb0, colb1, colb2, colb3)
    rows_ = (rowb0, rowb1, rowb2, rowb3)
    bufs = (bufA, bufB)
    semI = (semI0, semI1, semI2, semI3)
    semG = (semG0, semG1)
    semS = (semS0, semS1)
    base0 = wid * EPT
    t4 = lax.select(wid == NC * NS - 1, TAIL_CHUNKS // 4, CHUNKS // 4)

    def idx_issue(j, jm4):
        pltpu.async_copy(eidx_hbm.at[1, pl.ds(base0 + j * CH, CH)],
                         cols[jm4], semI[jm4])
        pltpu.async_copy(eidx_hbm.at[0, pl.ds(base0 + j * CH, CH)],
                         rows_[jm4], semI[jm4])

    def idx_wait(jm4):
        pltpu.make_async_copy(eidx_hbm.at[1, pl.ds(base0, CH)],
                              cols[jm4], semI[jm4]).wait()
        pltpu.make_async_copy(eidx_hbm.at[0, pl.ds(base0, CH)],
                              rows_[jm4], semI[jm4]).wait()

    def scat_wait(p2, r4):
        pltpu.make_async_copy(bufs[p2], acc.at[rows_[r4]], semS[p2]).wait()

    zero = jnp.zeros((16,), jnp.float32)

    def zbody(i, carry):
        for h in range(D // 16):
            bufA[i, pl.ds(h * 16, 16)] = zero
        return carry

    lax.fori_loop(0, CH, zbody, 0)
    off = 0
    for nrows in ROW_CHUNKS:
        pltpu.sync_copy(bufA.at[pl.ds(0, nrows)],
                        acc.at[pl.ds(s * ROWS_PER_TILE + off, nrows)])
        off += nrows

    @pl.when(wid < NC * NS - 1)
    def _():
        pltpu.sync_copy(ev_hbm.at[pl.ds(base0, EPT)], ev1)

    @pl.when(wid == NC * NS - 1)
    def _():
        pltpu.sync_copy(ev_hbm.at[pl.ds(base0, TAIL_E)],
                        ev1.at[pl.ds(0, TAIL_E)])

    plsc.subcore_barrier()
    idx_issue(0, 0)
    idx_issue(1, 1)
    idx_issue(2, 2)
    idx_wait(0)
    pltpu.async_copy(support_hbm.at[cols[0]], bufs[0], semG[0])

    def body(k2, carry):
        for cc in range(4):
            p4, p2 = (cc + 1) % 4, (cc + 1) % 2
            q4, q2 = cc, cc % 2
            r4 = (cc + 3) % 4
            cdyn = 4 * k2 + cc + 1

            if cc == 0:
                @pl.when(k2 >= 1)
                def _():
                    scat_wait(p2, r4)
            else:
                scat_wait(p2, r4)

            if cc == 0:
                idx_issue(cdyn + 2, r4)
            else:
                @pl.when(k2 < t4 - 1)
                def _():
                    idx_issue(cdyn + 2, r4)

            def _gather():
                idx_wait(p4)
                pltpu.async_copy(support_hbm.at[cols[p4]], bufs[p2],
                                 semG[p2])

            if cc == 3:
                @pl.when(k2 < t4 - 1)
                def _():
                    _gather()
            else:
                _gather()

            pltpu.make_async_copy(support_hbm.at[cols[q4]], bufs[q2],
                                  semG[q2]).wait()
            _scale_rows(bufs[q2], ev1, 4 * k2 + cc)
            pltpu.async_copy(bufs[q2], acc.at[rows_[q4]], semS[q2],
                             add=True)
        return carry

    lax.fori_loop(0, t4, body, 0)
    scat_wait(1, 3)
    plsc.subcore_barrier()

    off = 0
    for nrows in ROW_CHUNKS:
        r0 = s * ROWS_PER_TILE + off
        pltpu.sync_copy(acc.at[pl.ds(r0, nrows)],
                        out_hbm.at[c, pl.ds(r0, nrows)])
        off += nrows


_sc_scatter = functools.partial(
    pl.kernel,
    out_type=jax.ShapeDtypeStruct((NC, N_PAD, D), jnp.float32),
    mesh=plsc.VectorSubcoreMesh(core_axis_name="c", subcore_axis_name="s"),
    scratch_types=[
        pltpu.VMEM((CH,), jnp.int32),
        pltpu.VMEM((CH,), jnp.int32),
        pltpu.VMEM((CH,), jnp.int32),
        pltpu.VMEM((CH,), jnp.int32),
        pltpu.VMEM((CH,), jnp.int32),
        pltpu.VMEM((CH,), jnp.int32),
        pltpu.VMEM((CH,), jnp.int32),
        pltpu.VMEM((CH,), jnp.int32),
        pltpu.VMEM((EPT,), jnp.float32),
        pltpu.VMEM((CH, D), jnp.float32),
        pltpu.VMEM((CH, D), jnp.float32),
        pltpu.VMEM_SHARED((N_PAD, D), jnp.float32),
        pltpu.SemaphoreType.DMA,
        pltpu.SemaphoreType.DMA,
        pltpu.SemaphoreType.DMA,
        pltpu.SemaphoreType.DMA,
        pltpu.SemaphoreType.DMA,
        pltpu.SemaphoreType.DMA,
        pltpu.SemaphoreType.DMA,
        pltpu.SemaphoreType.DMA,
    ],
)(_sc_body)


def _comb_body(p_ref, b_ref, o_ref):
    o_ref[...] = p_ref[0] + p_ref[1] + b_ref[...]


def _combine(parts, b):
    return pl.pallas_call(
        _comb_body,
        grid=(10,),
        in_specs=[
            pl.BlockSpec((NC, N // 10, D), lambda i: (0, i, 0)),
            pl.BlockSpec((1, D), lambda i: (0, 0)),
        ],
        out_specs=pl.BlockSpec((N // 10, D), lambda i: (i, 0)),
        out_shape=jax.ShapeDtypeStruct((N, D), jnp.float32),
    )(parts, b.reshape(1, D))


def kernel(x, edge_index, edge_values, W, b):
    support = _matmul(x, W)
    parts = _sc_scatter(support, edge_index, edge_values)
    return _combine(parts, b)


# speedup vs baseline: 12.7244x; 1.0966x over previous
import functools

import jax
import jax.numpy as jnp
from jax import lax
from jax.experimental import pallas as pl
from jax.experimental.pallas import tpu as pltpu
from jax.experimental.pallas import tpu_sc as plsc

N = 10000
E = 320000
D = 128

NC = 2
NS = 16
CH = 64
CHUNKS = 160
EPT = CH * CHUNKS
FULL_TILES = E // EPT
TAIL_E = E - FULL_TILES * EPT
TAIL_CHUNKS = TAIL_E // CH
ROWS_PER_TILE = 632
N_PAD = ROWS_PER_TILE * NS
ROW_CHUNKS = (64,) * 9 + (56,)


def _mm_body(x_ref, w_ref, o_ref):
    o_ref[...] = jnp.dot(x_ref[...], w_ref[...],
                         preferred_element_type=jnp.float32)


def _matmul(x, W):
    return pl.pallas_call(
        _mm_body,
        grid=(5,),
        in_specs=[
            pl.BlockSpec((N // 5, D), lambda i: (i, 0)),
            pl.BlockSpec((D, D), lambda i: (0, 0)),
        ],
        out_specs=pl.BlockSpec((N // 5, D), lambda i: (i, 0)),
        out_shape=jax.ShapeDtypeStruct((N, D), jnp.float32),
    )(x, W)


def _bcast16(vec, j):
    return lax.gather(
        vec, jnp.full((16, 1), j, jnp.int32),
        lax.GatherDimensionNumbers(
            offset_dims=(), collapsed_slice_dims=(0,),
            start_index_map=(0,)),
        slice_sizes=(1,),
        mode=lax.GatherScatterMode.PROMISE_IN_BOUNDS)


def _scale_rows(buf, ev1, c):
    def gbody(g, carry):
        evg = ev1[pl.ds(c * CH + g * 16, 16)]
        for j in range(16):
            sc = _bcast16(evg, j)
            e = g * 16 + j
            for h in range(D // 16):
                buf[e, pl.ds(h * 16, 16)] = buf[e, pl.ds(h * 16, 16)] * sc
        return carry

    lax.fori_loop(0, CH // 16, gbody, 0)


def _sc_body(support_hbm, eidx_hbm, ev_hbm, out_hbm,
             colb0, colb1, colb2, colb3, colb4, colb5, colb6, colb7,
             rowb0, rowb1, rowb2, rowb3, rowb4, rowb5, rowb6, rowb7,
             ev1, bufA, bufB, bufC, bufD, acc,
             semI0, semI1, semI2, semI3, semI4, semI5, semI6, semI7,
             semG0, semG1, semG2, semG3, semS0, semS1, semS2, semS3):
    c_ax = lax.axis_index("c")
    s = lax.axis_index("s")
    wid = c_ax * NS + s
    cols = (colb0, colb1, colb2, colb3, colb4, colb5, colb6, colb7)
    rows_ = (rowb0, rowb1, rowb2, rowb3, rowb4, rowb5, rowb6, rowb7)
    bufs = (bufA, bufB, bufC, bufD)
    semI = (semI0, semI1, semI2, semI3, semI4, semI5, semI6, semI7)
    semG = (semG0, semG1, semG2, semG3)
    semS = (semS0, semS1, semS2, semS3)
    base0 = wid * EPT
    T = lax.select(wid == NC * NS - 1, TAIL_CHUNKS, CHUNKS)

    def idx_issue(j, m):
        pltpu.async_copy(eidx_hbm.at[1, pl.ds(base0 + j * CH, CH)],
                         cols[m], semI[m])
        pltpu.async_copy(eidx_hbm.at[0, pl.ds(base0 + j * CH, CH)],
                         rows_[m], semI[m])

    def idx_wait(m):
        pltpu.make_async_copy(eidx_hbm.at[1, pl.ds(base0, CH)],
                              cols[m], semI[m]).wait()
        pltpu.make_async_copy(eidx_hbm.at[0, pl.ds(base0, CH)],
                              rows_[m], semI[m]).wait()

    zero = jnp.zeros((16,), jnp.float32)

    def zbody(i, carry):
        for h in range(D // 16):
            bufA[i, pl.ds(h * 16, 16)] = zero
        return carry

    lax.fori_loop(0, CH, zbody, 0)
    off = 0
    for nrows in ROW_CHUNKS:
        pltpu.sync_copy(bufA.at[pl.ds(0, nrows)],
                        acc.at[pl.ds(s * ROWS_PER_TILE + off, nrows)])
        off += nrows

    @pl.when(wid < NC * NS - 1)
    def _():
        pltpu.sync_copy(ev_hbm.at[pl.ds(base0, EPT)], ev1)

    @pl.when(wid == NC * NS - 1)
    def _():
        pltpu.sync_copy(ev_hbm.at[pl.ds(base0, TAIL_E)],
                        ev1.at[pl.ds(0, TAIL_E)])

    plsc.subcore_barrier()
    for j in range(6):
        idx_issue(j, j)
    idx_wait(0)
    pltpu.async_copy(support_hbm.at[cols[0]], bufs[0], semG[0])
    idx_wait(1)
    pltpu.async_copy(support_hbm.at[cols[1]], bufs[1], semG[1])

    def body(k, carry):
        for u in range(8):
            c = k * 8 + u
            mW = (u + 6) % 8
            bW = (u + 2) % 4

            @pl.when(jnp.logical_and(c >= 2, c - 2 < T))
            def _():
                pltpu.make_async_copy(bufs[bW], acc.at[rows_[mW]],
                                      semS[bW]).wait()

            @pl.when(c + 6 < T)
            def _():
                idx_issue(c + 6, mW)

            @pl.when(c + 2 < T)
            def _():
                idx_wait((u + 2) % 8)
                pltpu.async_copy(support_hbm.at[cols[(u + 2) % 8]],
                                 bufs[bW], semG[bW])

            @pl.when(c < T)
            def _():
                pltpu.make_async_copy(support_hbm.at[cols[u]],
                                      bufs[u % 4], semG[u % 4]).wait()
                _scale_rows(bufs[u % 4], ev1, c)
                pltpu.async_copy(bufs[u % 4], acc.at[rows_[u]],
                                 semS[u % 4], add=True)
        return carry

    lax.fori_loop(0, (T + 9) // 8, body, 0)
    plsc.subcore_barrier()

    off = 0
    for nrows in ROW_CHUNKS:
        r0 = s * ROWS_PER_TILE + off
        pltpu.sync_copy(acc.at[pl.ds(r0, nrows)],
                        out_hbm.at[c_ax, pl.ds(r0, nrows)])
        off += nrows


_sc_scatter = functools.partial(
    pl.kernel,
    out_type=jax.ShapeDtypeStruct((NC, N_PAD, D), jnp.float32),
    mesh=plsc.VectorSubcoreMesh(core_axis_name="c", subcore_axis_name="s"),
    scratch_types=(
        [pltpu.VMEM((CH,), jnp.int32)] * 8
        + [pltpu.VMEM((CH,), jnp.int32)] * 8
        + [pltpu.VMEM((EPT,), jnp.float32)]
        + [pltpu.VMEM((CH, D), jnp.float32)] * 4
        + [pltpu.VMEM_SHARED((N_PAD, D), jnp.float32)]
        + [pltpu.SemaphoreType.DMA] * 8
        + [pltpu.SemaphoreType.DMA] * 4
        + [pltpu.SemaphoreType.DMA] * 4
    ),
)(_sc_body)


def _comb_body(p_ref, b_ref, o_ref):
    o_ref[...] = p_ref[0] + p_ref[1] + b_ref[...]


def _combine(parts, b):
    return pl.pallas_call(
        _comb_body,
        grid=(10,),
        in_specs=[
            pl.BlockSpec((NC, N // 10, D), lambda i: (0, i, 0)),
            pl.BlockSpec((1, D), lambda i: (0, 0)),
        ],
        out_specs=pl.BlockSpec((N // 10, D), lambda i: (i, 0)),
        out_shape=jax.ShapeDtypeStruct((N, D), jnp.float32),
    )(parts, b.reshape(1, D))


def kernel(x, edge_index, edge_values, W, b):
    support = _matmul(x, W)
    parts = _sc_scatter(support, edge_index, edge_values)
    return _combine(parts, b)


# speedup vs baseline: 12.9006x; 1.0138x over previous
import functools

import jax
import jax.numpy as jnp
from jax import lax
from jax.experimental import pallas as pl
from jax.experimental.pallas import tpu as pltpu
from jax.experimental.pallas import tpu_sc as plsc

N = 10000
E = 320000
D = 128

NC = 2
NS = 16
CH = 64
CHUNKS_LO = (E // CH) // (NC * NS)
HEAVY = (E // CH) - CHUNKS_LO * NC * NS
EPT_MAX = CH * (CHUNKS_LO + 1)
ROWS_PER_TILE = 632
N_PAD = ROWS_PER_TILE * NS
ROW_CHUNKS = (64,) * 9 + (56,)


def _mm_body(x_ref, w_ref, o_ref):
    o_ref[...] = jnp.dot(x_ref[...], w_ref[...],
                         preferred_element_type=jnp.float32)


def _matmul(x, W):
    return pl.pallas_call(
        _mm_body,
        grid=(5,),
        in_specs=[
            pl.BlockSpec((N // 5, D), lambda i: (i, 0)),
            pl.BlockSpec((D, D), lambda i: (0, 0)),
        ],
        out_specs=pl.BlockSpec((N // 5, D), lambda i: (i, 0)),
        out_shape=jax.ShapeDtypeStruct((N, D), jnp.float32),
    )(x, W)


def _bcast16(vec, j):
    return lax.gather(
        vec, jnp.full((16, 1), j, jnp.int32),
        lax.GatherDimensionNumbers(
            offset_dims=(), collapsed_slice_dims=(0,),
            start_index_map=(0,)),
        slice_sizes=(1,),
        mode=lax.GatherScatterMode.PROMISE_IN_BOUNDS)


def _scale_rows(buf, ev1, c):
    def gbody(g, carry):
        evg = ev1[pl.ds(c * CH + g * 16, 16)]
        for j in range(16):
            sc = _bcast16(evg, j)
            e = g * 16 + j
            for h in range(D // 16):
                buf[e, pl.ds(h * 16, 16)] = buf[e, pl.ds(h * 16, 16)] * sc
        return carry

    lax.fori_loop(0, CH // 16, gbody, 0)


def _sc_body(support_hbm, eidx_hbm, ev_hbm, out_hbm,
             colb0, colb1, colb2, colb3, colb4, colb5, colb6, colb7,
             rowb0, rowb1, rowb2, rowb3, rowb4, rowb5, rowb6, rowb7,
             ev1, bufA, bufB, bufC, bufD, acc,
             semI0, semI1, semI2, semI3, semI4, semI5, semI6, semI7,
             semG0, semG1, semG2, semG3, semS0, semS1, semS2, semS3):
    c_ax = lax.axis_index("c")
    s = lax.axis_index("s")
    wid = c_ax * NS + s
    cols = (colb0, colb1, colb2, colb3, colb4, colb5, colb6, colb7)
    rows_ = (rowb0, rowb1, rowb2, rowb3, rowb4, rowb5, rowb6, rowb7)
    bufs = (bufA, bufB, bufC, bufD)
    semI = (semI0, semI1, semI2, semI3, semI4, semI5, semI6, semI7)
    semG = (semG0, semG1, semG2, semG3)
    semS = (semS0, semS1, semS2, semS3)
    base0 = CH * (CHUNKS_LO * wid + jnp.minimum(wid, HEAVY))
    T = lax.select(wid < HEAVY, CHUNKS_LO + 1, CHUNKS_LO)

    def idx_issue(j, m):
        pltpu.async_copy(eidx_hbm.at[1, pl.ds(base0 + j * CH, CH)],
                         cols[m], semI[m])
        pltpu.async_copy(eidx_hbm.at[0, pl.ds(base0 + j * CH, CH)],
                         rows_[m], semI[m])

    def idx_wait(m):
        pltpu.make_async_copy(eidx_hbm.at[1, pl.ds(base0, CH)],
                              cols[m], semI[m]).wait()
        pltpu.make_async_copy(eidx_hbm.at[0, pl.ds(base0, CH)],
                              rows_[m], semI[m]).wait()

    zero = jnp.zeros((16,), jnp.float32)

    def zbody(i, carry):
        for h in range(D // 16):
            bufA[i, pl.ds(h * 16, 16)] = zero
        return carry

    lax.fori_loop(0, CH, zbody, 0)
    off = 0
    for nrows in ROW_CHUNKS:
        pltpu.sync_copy(bufA.at[pl.ds(0, nrows)],
                        acc.at[pl.ds(s * ROWS_PER_TILE + off, nrows)])
        off += nrows

    @pl.when(wid < HEAVY)
    def _():
        pltpu.sync_copy(ev_hbm.at[pl.ds(base0, EPT_MAX)], ev1)

    @pl.when(wid >= HEAVY)
    def _():
        pltpu.sync_copy(ev_hbm.at[pl.ds(base0, CH * CHUNKS_LO)],
                        ev1.at[pl.ds(0, CH * CHUNKS_LO)])

    plsc.subcore_barrier()
    for j in range(6):
        idx_issue(j, j)
    idx_wait(0)
    pltpu.async_copy(support_hbm.at[cols[0]], bufs[0], semG[0])
    idx_wait(1)
    pltpu.async_copy(support_hbm.at[cols[1]], bufs[1], semG[1])

    def body(k, carry):
        for u in range(8):
            c = k * 8 + u
            mW = (u + 6) % 8
            bW = (u + 2) % 4

            @pl.when(jnp.logical_and(c >= 2, c - 2 < T))
            def _():
                pltpu.make_async_copy(bufs[bW], acc.at[rows_[mW]],
                                      semS[bW]).wait()

            @pl.when(c + 6 < T)
            def _():
                idx_issue(c + 6, mW)

            @pl.when(c + 2 < T)
            def _():
                idx_wait((u + 2) % 8)
                pltpu.async_copy(support_hbm.at[cols[(u + 2) % 8]],
                                 bufs[bW], semG[bW])

            @pl.when(c < T)
            def _():
                pltpu.make_async_copy(support_hbm.at[cols[u]],
                                      bufs[u % 4], semG[u % 4]).wait()
                _scale_rows(bufs[u % 4], ev1, c)
                pltpu.async_copy(bufs[u % 4], acc.at[rows_[u]],
                                 semS[u % 4], add=True)
        return carry

    lax.fori_loop(0, (T + 9) // 8, body, 0)
    plsc.subcore_barrier()

    off = 0
    for nrows in ROW_CHUNKS:
        r0 = s * ROWS_PER_TILE + off
        pltpu.sync_copy(acc.at[pl.ds(r0, nrows)],
                        out_hbm.at[c_ax, pl.ds(r0, nrows)])
        off += nrows


_sc_scatter = functools.partial(
    pl.kernel,
    out_type=jax.ShapeDtypeStruct((NC, N_PAD, D), jnp.float32),
    mesh=plsc.VectorSubcoreMesh(core_axis_name="c", subcore_axis_name="s"),
    scratch_types=(
        [pltpu.VMEM((CH,), jnp.int32)] * 8
        + [pltpu.VMEM((CH,), jnp.int32)] * 8
        + [pltpu.VMEM((EPT_MAX,), jnp.float32)]
        + [pltpu.VMEM((CH, D), jnp.float32)] * 4
        + [pltpu.VMEM_SHARED((N_PAD, D), jnp.float32)]
        + [pltpu.SemaphoreType.DMA] * 8
        + [pltpu.SemaphoreType.DMA] * 4
        + [pltpu.SemaphoreType.DMA] * 4
    ),
)(_sc_body)


def _comb_body(p_ref, b_ref, o_ref):
    o_ref[...] = p_ref[0] + p_ref[1] + b_ref[...]


def _combine(parts, b):
    return pl.pallas_call(
        _comb_body,
        grid=(10,),
        in_specs=[
            pl.BlockSpec((NC, N // 10, D), lambda i: (0, i, 0)),
            pl.BlockSpec((1, D), lambda i: (0, 0)),
        ],
        out_specs=pl.BlockSpec((N // 10, D), lambda i: (i, 0)),
        out_shape=jax.ShapeDtypeStruct((N, D), jnp.float32),
    )(parts, b.reshape(1, D))


def kernel(x, edge_index, edge_values, W, b):
    support = _matmul(x, W)
    parts = _sc_scatter(support, edge_index, edge_values)
    return _combine(parts, b)
